# baseline (speedup 1.0000x reference)
import functools

import jax
import jax.numpy as jnp
from jax import lax
from jax.experimental import pallas as pl
from jax.experimental.pallas import tpu as pltpu
from jax.experimental.pallas import tpu_sc as plsc

NC = 2
NS = 16
NW = NC * NS
LANES = 16


def _sc_agg(src, dst, table=None, hist_shape=None):
    gather = table is not None
    if gather:
        N, D = table.shape
    else:
        N, D = hist_shape
    E = dst.shape[0]
    assert E % NW == 0
    EW = E // NW
    C = 400
    assert EW % C == 0
    ng = EW // C
    assert N % NS == 0
    TR = N // NS

    mesh = plsc.VectorSubcoreMesh(core_axis_name="c", subcore_axis_name="s")

    src3 = src.reshape(NW, ng, C) if gather else None
    dst3 = dst.reshape(NW, ng, C)

    scratch = [
        pltpu.VMEM((ng, C), jnp.int32),
        pltpu.VMEM((ng, C), jnp.int32),
        pltpu.VMEM((2, C, D), jnp.float32),
        pltpu.VMEM_SHARED((N, D), jnp.float32),
        pltpu.SemaphoreType.DMA,
        pltpu.SemaphoreType.DMA,
    ]

    def body(table_h, src3_h, dst3_h, out_h, sidx, didx, bufs, acc, gsem, ssem):
        c = lax.axis_index("c")
        s = lax.axis_index("s")
        w = c * NS + s

        nv = D // LANES

        def fill_buf0(val):
            def fb(t, carry):
                r = t // nv
                col = (t % nv) * LANES
                bufs[0, r, pl.ds(col, LANES)] = jnp.full(
                    (LANES,), val, jnp.float32)
                return carry
            lax.fori_loop(0, C * nv, fb, 0)

        fill_buf0(0.0)
        r0 = s * TR
        off = 0
        while off < TR:
            m = min(C, TR - off)
            pltpu.sync_copy(bufs.at[0, pl.ds(0, m)],
                            acc.at[pl.ds(r0 + off, m)])
            off += m

        pltpu.sync_copy(dst3_h.at[w], didx)
        if gather:
            pltpu.sync_copy(src3_h.at[w], sidx)
        else:
            fill_buf0(1.0)
        plsc.subcore_barrier()

        if gather:
            def start_gather(g, p):
                pltpu.async_copy(table_h.at[sidx.at[g]], bufs.at[p], gsem)

            start_gather(0, 0)

            def group(t, carry):
                p = lax.rem(t, 2)
                q = 1 - p
                pltpu.make_async_copy(
                    table_h.at[pl.ds(0, C)], bufs.at[p], gsem).wait()
                pltpu.async_copy(bufs.at[p], acc.at[didx.at[t]], ssem,
                                 add=True)

                @pl.when(t + 1 < ng)
                def _():
                    start_gather(t + 1, q)

                pltpu.make_async_copy(
                    bufs.at[p], acc.at[pl.ds(0, C)], ssem).wait()
                return carry

            lax.fori_loop(0, ng, group, 0)
        else:
            def group(t, carry):
                pltpu.async_copy(bufs.at[0], acc.at[didx.at[t]], ssem,
                                 add=True)
                pltpu.make_async_copy(
                    bufs.at[0], acc.at[pl.ds(0, C)], ssem).wait()
                return carry

            lax.fori_loop(0, ng, group, 0)

        plsc.subcore_barrier()
        pltpu.sync_copy(acc.at[pl.ds(r0, TR)], out_h.at[c, s])

    if gather:
        args = (table, src3, dst3)

        def k_gather(table_h, src3_h, dst3_h, out_h,
                     sidx, didx, bufs, acc, gsem, ssem):
            body(table_h, src3_h, dst3_h, out_h,
                 sidx, didx, bufs, acc, gsem, ssem)

        fn = k_gather
    else:
        args = (dst3,)

        def k_hist(dst3_h, out_h, sidx, didx, bufs, acc, gsem, ssem):
            body(None, None, dst3_h, out_h,
                 sidx, didx, bufs, acc, gsem, ssem)

        fn = k_hist

    run = functools.partial(
        pl.kernel,
        mesh=mesh,
        out_type=jax.ShapeDtypeStruct((NC, NS, TR, D), jnp.float32),
        scratch_types=scratch,
        compiler_params=pltpu.CompilerParams(use_tc_tiling_on_sc=False),
    )(fn)
    return run(*args).reshape(NC, N, D)


_ROWS = 1000


def _dis_of(degp_ref):
    deg = degp_ref[0, :, 0] + degp_ref[1, :, 0] + 1.0
    return lax.rsqrt(deg)[:, None]


def _prep_body(degp_ref, x_ref, w_ref, o1_ref, o2_ref):
    dis = _dis_of(degp_ref)
    h = jnp.dot(x_ref[...], w_ref[...],
                preferred_element_type=jnp.float32) * dis
    half = h.shape[1] // 2
    o1_ref[...] = h[:, :half]
    o2_ref[...] = h[:, half:]


def _tc_prep(degp, x, W):
    N, Din = x.shape
    Dh = W.shape[1]
    half = Dh // 2
    return pl.pallas_call(
        _prep_body,
        grid=(N // _ROWS,),
        in_specs=[
            pl.BlockSpec((NC, _ROWS, LANES), lambda i: (0, i, 0)),
            pl.BlockSpec((_ROWS, Din), lambda i: (i, 0)),
            pl.BlockSpec((Din, Dh), lambda i: (0, 0)),
        ],
        out_specs=[
            pl.BlockSpec((_ROWS, half), lambda i: (i, 0)),
            pl.BlockSpec((_ROWS, half), lambda i: (i, 0)),
        ],
        out_shape=[
            jax.ShapeDtypeStruct((N, half), jnp.float32),
            jax.ShapeDtypeStruct((N, half), jnp.float32),
        ],
    )(degp, x, W)


def _fp1_body(aggA_ref, aggB_ref, hpA_ref, hpB_ref, degp_ref, b_ref, w_ref,
              o_ref):
    dis = _dis_of(degp_ref)
    half = hpA_ref.shape[1]
    aA = aggA_ref[0] + aggA_ref[1] + hpA_ref[...]
    aB = aggB_ref[0] + aggB_ref[1] + hpB_ref[...]
    yA = jnp.maximum(dis * aA + b_ref[:, :half], 0.0)
    yB = jnp.maximum(dis * aB + b_ref[:, half:], 0.0)
    y = jnp.concatenate([yA, yB], axis=1)
    o_ref[...] = jnp.dot(y, w_ref[...],
                         preferred_element_type=jnp.float32) * dis


def _tc_finish_prep1(aggA, aggB, hpA, hpB, degp, b, W):
    N, half = hpA.shape
    D = 2 * half
    K = W.shape[1]
    return pl.pallas_call(
        _fp1_body,
        grid=(N // _ROWS,),
        in_specs=[
            pl.BlockSpec((NC, _ROWS, half), lambda i: (0, i, 0)),
            pl.BlockSpec((NC, _ROWS, half), lambda i: (0, i, 0)),
            pl.BlockSpec((_ROWS, half), lambda i: (i, 0)),
            pl.BlockSpec((_ROWS, half), lambda i: (i, 0)),
            pl.BlockSpec((NC, _ROWS, LANES), lambda i: (0, i, 0)),
            pl.BlockSpec((1, D), lambda i: (0, 0)),
            pl.BlockSpec((D, K), lambda i: (0, 0)),
        ],
        out_specs=pl.BlockSpec((_ROWS, K), lambda i: (i, 0)),
        out_shape=jax.ShapeDtypeStruct((N, K), jnp.float32),
    )(aggA, aggB, hpA, hpB, degp, b.reshape(1, D), W)


def _fp_body(aggp_ref, hp_ref, degp_ref, b_ref, w_ref, o_ref):
    dis = _dis_of(degp_ref)
    a = aggp_ref[0] + aggp_ref[1] + hp_ref[...]
    y = jnp.maximum(dis * a + b_ref[...], 0.0)
    o_ref[...] = jnp.dot(y, w_ref[...],
                         preferred_element_type=jnp.float32) * dis


def _tc_finish_prep(aggp, hp, degp, b, W):
    N, D = hp.shape
    K = W.shape[1]
    return pl.pallas_call(
        _fp_body,
        grid=(N // _ROWS,),
        in_specs=[
            pl.BlockSpec((NC, _ROWS, D), lambda i: (0, i, 0)),
            pl.BlockSpec((_ROWS, D), lambda i: (i, 0)),
            pl.BlockSpec((NC, _ROWS, LANES), lambda i: (0, i, 0)),
            pl.BlockSpec((1, D), lambda i: (0, 0)),
            pl.BlockSpec((D, K), lambda i: (0, 0)),
        ],
        out_specs=pl.BlockSpec((_ROWS, K), lambda i: (i, 0)),
        out_shape=jax.ShapeDtypeStruct((N, K), jnp.float32),
    )(aggp, hp, degp, b.reshape(1, D), W)


def _final_body(aggp_ref, hp_ref, degp_ref, b_ref, o_ref):
    dis = _dis_of(degp_ref)
    a = aggp_ref[0] + aggp_ref[1] + hp_ref[...]
    y = dis * a + b_ref[0, 0]
    o_ref[...] = y[:, :1]


def _tc_final(aggp, hp, degp, b):
    N = hp.shape[0]
    return pl.pallas_call(
        _final_body,
        grid=(N // _ROWS,),
        in_specs=[
            pl.BlockSpec((NC, _ROWS, LANES), lambda i: (0, i, 0)),
            pl.BlockSpec((_ROWS, LANES), lambda i: (i, 0)),
            pl.BlockSpec((NC, _ROWS, LANES), lambda i: (0, i, 0)),
            pl.BlockSpec((1, 1), lambda i: (0, 0)),
        ],
        out_specs=pl.BlockSpec((_ROWS, 1), lambda i: (i, 0)),
        out_shape=jax.ShapeDtypeStruct((N, 1), jnp.float32),
    )(aggp, hp, degp, b.reshape(1, 1))


def kernel(x, edge_list, W1, b1, W2, b2, W3, b3):
    N = x.shape[0]
    src = edge_list[0]
    dst = edge_list[1]
    W3p = jnp.pad(W3, ((0, 0), (0, LANES - W3.shape[1])))
    b3p = jnp.pad(b3, (0, 0))

    degp = _sc_agg(src, dst, hist_shape=(N, LANES))
    h1pA, h1pB = _tc_prep(degp, x, W1)
    agg1A = _sc_agg(src, dst, table=h1pA)
    agg1B = _sc_agg(src, dst, table=h1pB)
    h2p = _tc_finish_prep1(agg1A, agg1B, h1pA, h1pB, degp, b1, W2)
    agg2 = _sc_agg(src, dst, table=h2p)
    h3p = _tc_finish_prep(agg2, h2p, degp, b2, W3p)
    agg3 = _sc_agg(src, dst, table=h3p)
    return _tc_final(agg3, h3p, degp, b3p)


# speedup vs baseline: 1.2269x; 1.2269x over previous
import functools

import jax
import jax.numpy as jnp
from jax import lax
from jax.experimental import pallas as pl
from jax.experimental.pallas import tpu as pltpu
from jax.experimental.pallas import tpu_sc as plsc

NC = 2
NS = 16
NW = NC * NS
LANES = 16


def _sc_agg(src, dst, table=None, hist_shape=None):
    gather = table is not None
    if gather:
        N, D = table.shape
        dtype = table.dtype
    else:
        N, D = hist_shape
        dtype = jnp.float32
    E = dst.shape[0]
    assert E % NW == 0
    EW = E // NW
    C = 80
    NB = 5
    assert EW % (C * NB) == 0
    nch = EW // C
    ng = nch // NB
    assert N % NS == 0
    TR = N // NS
    lanes = LANES * 4 // jnp.dtype(dtype).itemsize
    nv = D // lanes

    mesh = plsc.VectorSubcoreMesh(core_axis_name="c", subcore_axis_name="s")

    src3 = src.reshape(NW, nch, C) if gather else None
    dst3 = dst.reshape(NW, nch, C)

    scratch = [
        pltpu.VMEM((nch, C), jnp.int32),
        pltpu.VMEM((nch, C), jnp.int32),
        pltpu.VMEM((2, NB, C, D), dtype),
        pltpu.VMEM_SHARED((N, D), dtype),
        pltpu.SemaphoreType.DMA,
        pltpu.SemaphoreType.DMA,
    ]

    def body(table_h, src3_h, dst3_h, out_h, sidx, didx, bufs, acc, gsem, ssem):
        c = lax.axis_index("c")
        s = lax.axis_index("s")
        w = c * NS + s

        def fill_buf0(val):
            def fb(t, carry):
                r = t // nv
                col = (t % nv) * lanes
                bufs[0, 0, r, pl.ds(col, lanes)] = jnp.full(
                    (lanes,), val, dtype)
                return carry
            lax.fori_loop(0, C * nv, fb, 0)

        fill_buf0(0.0)
        r0 = s * TR
        off = 0
        while off < TR:
            m = min(C, TR - off)
            pltpu.sync_copy(bufs.at[0, 0, pl.ds(0, m)],
                            acc.at[pl.ds(r0 + off, m)])
            off += m

        pltpu.sync_copy(dst3_h.at[w], didx)
        if gather:
            pltpu.sync_copy(src3_h.at[w], sidx)
        else:
            fill_buf0(1.0)
        plsc.subcore_barrier()

        if gather:
            def start_gathers(g, p):
                for j in range(NB):
                    pltpu.async_copy(table_h.at[sidx.at[g * NB + j]],
                                     bufs.at[p, j], gsem)

            def drain(sem, p):
                for j in range(NB):
                    pltpu.make_async_copy(
                        table_h.at[pl.ds(0, C)], bufs.at[p, j], sem
                    ).wait()

            start_gathers(0, 0)

            def group(t, carry):
                p = lax.rem(t, 2)
                q = 1 - p
                drain(gsem, p)
                for j in range(NB):
                    pltpu.async_copy(bufs.at[p, j],
                                     acc.at[didx.at[t * NB + j]],
                                     ssem, add=True)

                @pl.when(t + 1 < ng)
                def _():
                    start_gathers(t + 1, q)

                drain(ssem, p)
                return carry

            lax.fori_loop(0, ng, group, 0)
        else:
            def group(t, carry):
                for j in range(NB):
                    pltpu.async_copy(bufs.at[0, 0],
                                     acc.at[didx.at[t * NB + j]],
                                     ssem, add=True)
                for j in range(NB):
                    pltpu.make_async_copy(
                        bufs.at[0, 0], acc.at[pl.ds(0, C)], ssem
                    ).wait()
                return carry

            lax.fori_loop(0, ng, group, 0)

        plsc.subcore_barrier()
        pltpu.sync_copy(acc.at[pl.ds(r0, TR)], out_h.at[c, s])

    if gather:
        args = (table, src3, dst3)

        def k_gather(table_h, src3_h, dst3_h, out_h,
                     sidx, didx, bufs, acc, gsem, ssem):
            body(table_h, src3_h, dst3_h, out_h,
                 sidx, didx, bufs, acc, gsem, ssem)

        fn = k_gather
    else:
        args = (dst3,)

        def k_hist(dst3_h, out_h, sidx, didx, bufs, acc, gsem, ssem):
            body(None, None, dst3_h, out_h,
                 sidx, didx, bufs, acc, gsem, ssem)

        fn = k_hist

    run = functools.partial(
        pl.kernel,
        mesh=mesh,
        out_type=jax.ShapeDtypeStruct((NC, NS, TR, D), dtype),
        scratch_types=scratch,
        compiler_params=pltpu.CompilerParams(use_tc_tiling_on_sc=False),
    )(fn)
    return run(*args).reshape(NC, N, D)


_ROWS = 1000


def _dis_of(degp_ref):
    deg = degp_ref[0, :, 0] + degp_ref[1, :, 0] + 1.0
    return lax.rsqrt(deg)[:, None]


def _prep_body(degp_ref, x_ref, w_ref, o1_ref, o2_ref, o1b_ref, o2b_ref):
    dis = _dis_of(degp_ref)
    h = jnp.dot(x_ref[...], w_ref[...],
                preferred_element_type=jnp.float32) * dis
    half = h.shape[1] // 2
    o1_ref[...] = h[:, :half]
    o2_ref[...] = h[:, half:]
    o1b_ref[...] = h[:, :half].astype(jnp.bfloat16)
    o2b_ref[...] = h[:, half:].astype(jnp.bfloat16)


def _tc_prep(degp, x, W):
    N, Din = x.shape
    Dh = W.shape[1]
    half = Dh // 2
    return pl.pallas_call(
        _prep_body,
        grid=(N // _ROWS,),
        in_specs=[
            pl.BlockSpec((NC, _ROWS, LANES), lambda i: (0, i, 0)),
            pl.BlockSpec((_ROWS, Din), lambda i: (i, 0)),
            pl.BlockSpec((Din, Dh), lambda i: (0, 0)),
        ],
        out_specs=[
            pl.BlockSpec((_ROWS, half), lambda i: (i, 0)),
            pl.BlockSpec((_ROWS, half), lambda i: (i, 0)),
            pl.BlockSpec((_ROWS, half), lambda i: (i, 0)),
            pl.BlockSpec((_ROWS, half), lambda i: (i, 0)),
        ],
        out_shape=[
            jax.ShapeDtypeStruct((N, half), jnp.float32),
            jax.ShapeDtypeStruct((N, half), jnp.float32),
            jax.ShapeDtypeStruct((N, half), jnp.bfloat16),
            jax.ShapeDtypeStruct((N, half), jnp.bfloat16),
        ],
    )(degp, x, W)


def _fp1_body(aggA_ref, aggB_ref, hpA_ref, hpB_ref, degp_ref, b_ref, w_ref,
              o_ref, ob_ref):
    dis = _dis_of(degp_ref)
    half = hpA_ref.shape[1]
    aA = (aggA_ref[0].astype(jnp.float32) + aggA_ref[1].astype(jnp.float32)
          + hpA_ref[...])
    aB = (aggB_ref[0].astype(jnp.float32) + aggB_ref[1].astype(jnp.float32)
          + hpB_ref[...])
    yA = jnp.maximum(dis * aA + b_ref[:, :half], 0.0)
    yB = jnp.maximum(dis * aB + b_ref[:, half:], 0.0)
    y = jnp.concatenate([yA, yB], axis=1)
    h = jnp.dot(y, w_ref[...], preferred_element_type=jnp.float32) * dis
    o_ref[...] = h
    ob_ref[...] = h.astype(jnp.bfloat16)


def _tc_finish_prep1(aggA, aggB, hpA, hpB, degp, b, W):
    N, half = hpA.shape
    D = 2 * half
    K = W.shape[1]
    return pl.pallas_call(
        _fp1_body,
        grid=(N // _ROWS,),
        in_specs=[
            pl.BlockSpec((NC, _ROWS, half), lambda i: (0, i, 0)),
            pl.BlockSpec((NC, _ROWS, half), lambda i: (0, i, 0)),
            pl.BlockSpec((_ROWS, half), lambda i: (i, 0)),
            pl.BlockSpec((_ROWS, half), lambda i: (i, 0)),
            pl.BlockSpec((NC, _ROWS, LANES), lambda i: (0, i, 0)),
            pl.BlockSpec((1, D), lambda i: (0, 0)),
            pl.BlockSpec((D, K), lambda i: (0, 0)),
        ],
        out_specs=[
            pl.BlockSpec((_ROWS, K), lambda i: (i, 0)),
            pl.BlockSpec((_ROWS, K), lambda i: (i, 0)),
        ],
        out_shape=[
            jax.ShapeDtypeStruct((N, K), jnp.float32),
            jax.ShapeDtypeStruct((N, K), jnp.bfloat16),
        ],
    )(aggA, aggB, hpA, hpB, degp, b.reshape(1, D), W)


def _fp_body(aggp_ref, hp_ref, degp_ref, b_ref, w_ref, o_ref):
    dis = _dis_of(degp_ref)
    a = (aggp_ref[0].astype(jnp.float32) + aggp_ref[1].astype(jnp.float32)
         + hp_ref[...])
    y = jnp.maximum(dis * a + b_ref[...], 0.0)
    o_ref[...] = jnp.dot(y, w_ref[...],
                         preferred_element_type=jnp.float32) * dis


def _tc_finish_prep(aggp, hp, degp, b, W):
    N, D = hp.shape
    K = W.shape[1]
    return pl.pallas_call(
        _fp_body,
        grid=(N // _ROWS,),
        in_specs=[
            pl.BlockSpec((NC, _ROWS, D), lambda i: (0, i, 0)),
            pl.BlockSpec((_ROWS, D), lambda i: (i, 0)),
            pl.BlockSpec((NC, _ROWS, LANES), lambda i: (0, i, 0)),
            pl.BlockSpec((1, D), lambda i: (0, 0)),
            pl.BlockSpec((D, K), lambda i: (0, 0)),
        ],
        out_specs=pl.BlockSpec((_ROWS, K), lambda i: (i, 0)),
        out_shape=jax.ShapeDtypeStruct((N, K), jnp.float32),
    )(aggp, hp, degp, b.reshape(1, D), W)


def _final_body(aggp_ref, hp_ref, degp_ref, b_ref, o_ref):
    dis = _dis_of(degp_ref)
    a = aggp_ref[0] + aggp_ref[1] + hp_ref[...]
    y = dis * a + b_ref[0, 0]
    o_ref[...] = y[:, :1]


def _tc_final(aggp, hp, degp, b):
    N = hp.shape[0]
    return pl.pallas_call(
        _final_body,
        grid=(N // _ROWS,),
        in_specs=[
            pl.BlockSpec((NC, _ROWS, LANES), lambda i: (0, i, 0)),
            pl.BlockSpec((_ROWS, LANES), lambda i: (i, 0)),
            pl.BlockSpec((NC, _ROWS, LANES), lambda i: (0, i, 0)),
            pl.BlockSpec((1, 1), lambda i: (0, 0)),
        ],
        out_specs=pl.BlockSpec((_ROWS, 1), lambda i: (i, 0)),
        out_shape=jax.ShapeDtypeStruct((N, 1), jnp.float32),
    )(aggp, hp, degp, b.reshape(1, 1))


def kernel(x, edge_list, W1, b1, W2, b2, W3, b3):
    N = x.shape[0]
    src = edge_list[0]
    dst = edge_list[1]
    W3p = jnp.pad(W3, ((0, 0), (0, LANES - W3.shape[1])))

    degp = _sc_agg(src, dst, hist_shape=(N, LANES))
    h1pA, h1pB, h1pAb, h1pBb = _tc_prep(degp, x, W1)
    agg1A = _sc_agg(src, dst, table=h1pAb)
    agg1B = _sc_agg(src, dst, table=h1pBb)
    h2p, h2pb = _tc_finish_prep1(agg1A, agg1B, h1pA, h1pB, degp, b1, W2)
    agg2 = _sc_agg(src, dst, table=h2pb)
    h3p = _tc_finish_prep(agg2, h2p, degp, b2, W3p)
    agg3 = _sc_agg(src, dst, table=h3p)
    return _tc_final(agg3, h3p, degp, b3)


# speedup vs baseline: 1.2291x; 1.0018x over previous
import functools

import jax
import jax.numpy as jnp
from jax import lax
from jax.experimental import pallas as pl
from jax.experimental.pallas import tpu as pltpu
from jax.experimental.pallas import tpu_sc as plsc

NC = 2
NS = 16
NW = NC * NS
LANES = 16


def _sc_agg(src, dst, table=None, hist_shape=None):
    gather = table is not None
    if gather:
        N, D = table.shape
        dtype = table.dtype
    else:
        N, D = hist_shape
        dtype = jnp.float32
    E = dst.shape[0]
    assert E % NW == 0
    EW = E // NW
    C = 80
    NB = 5
    assert EW % (C * NB) == 0
    nch = EW // C
    ng = nch // NB
    assert N % NS == 0
    TR = N // NS
    lanes = LANES * 4 // jnp.dtype(dtype).itemsize
    nv = D // lanes

    mesh = plsc.VectorSubcoreMesh(core_axis_name="c", subcore_axis_name="s")

    src3 = src.reshape(NW, nch, C) if gather else None
    dst3 = dst.reshape(NW, nch, C)

    scratch = [
        pltpu.VMEM((nch, C), jnp.int32),
        pltpu.VMEM((nch, C), jnp.int32),
        pltpu.VMEM((2, NB, C, D), dtype),
        pltpu.VMEM_SHARED((N, D), dtype),
        pltpu.SemaphoreType.DMA,
        pltpu.SemaphoreType.DMA,
    ]

    def body(table_h, src3_h, dst3_h, out_h, sidx, didx, bufs, acc, gsem, ssem):
        c = lax.axis_index("c")
        s = lax.axis_index("s")
        w = c * NS + s

        def fill_buf0(val):
            def fb(t, carry):
                r = t // nv
                col = (t % nv) * lanes
                bufs[0, 0, r, pl.ds(col, lanes)] = jnp.full(
                    (lanes,), val, dtype)
                return carry
            lax.fori_loop(0, C * nv, fb, 0)

        fill_buf0(0.0)
        r0 = s * TR
        off = 0
        while off < TR:
            m = min(C, TR - off)
            pltpu.sync_copy(bufs.at[0, 0, pl.ds(0, m)],
                            acc.at[pl.ds(r0 + off, m)])
            off += m

        pltpu.sync_copy(dst3_h.at[w], didx)
        if gather:
            pltpu.sync_copy(src3_h.at[w], sidx)
        else:
            fill_buf0(1.0)
        plsc.subcore_barrier()

        if gather:
            def start_gathers(g, p):
                for j in range(NB):
                    pltpu.async_copy(table_h.at[sidx.at[g * NB + j]],
                                     bufs.at[p, j], gsem)

            def drain(sem, p):
                for j in range(NB):
                    pltpu.make_async_copy(
                        table_h.at[pl.ds(0, C)], bufs.at[p, j], sem
                    ).wait()

            start_gathers(0, 0)

            def group(t, carry):
                p = lax.rem(t, 2)
                q = 1 - p
                drain(gsem, p)
                for j in range(NB):
                    pltpu.async_copy(bufs.at[p, j],
                                     acc.at[didx.at[t * NB + j]],
                                     ssem, add=True)

                @pl.when(t + 1 < ng)
                def _():
                    start_gathers(t + 1, q)

                drain(ssem, p)
                return carry

            lax.fori_loop(0, ng, group, 0)
        else:
            def group(t, carry):
                for j in range(NB):
                    pltpu.async_copy(bufs.at[0, 0],
                                     acc.at[didx.at[t * NB + j]],
                                     ssem, add=True)
                for j in range(NB):
                    pltpu.make_async_copy(
                        bufs.at[0, 0], acc.at[pl.ds(0, C)], ssem
                    ).wait()
                return carry

            lax.fori_loop(0, ng, group, 0)

        plsc.subcore_barrier()
        pltpu.sync_copy(acc.at[pl.ds(r0, TR)], out_h.at[c, s])

    if gather:
        args = (table, src3, dst3)

        def k_gather(table_h, src3_h, dst3_h, out_h,
                     sidx, didx, bufs, acc, gsem, ssem):
            body(table_h, src3_h, dst3_h, out_h,
                 sidx, didx, bufs, acc, gsem, ssem)

        fn = k_gather
    else:
        args = (dst3,)

        def k_hist(dst3_h, out_h, sidx, didx, bufs, acc, gsem, ssem):
            body(None, None, dst3_h, out_h,
                 sidx, didx, bufs, acc, gsem, ssem)

        fn = k_hist

    run = functools.partial(
        pl.kernel,
        mesh=mesh,
        out_type=jax.ShapeDtypeStruct((NC, NS, TR, D), dtype),
        scratch_types=scratch,
        compiler_params=pltpu.CompilerParams(use_tc_tiling_on_sc=False),
    )(fn)
    return run(*args).reshape(NC, N, D)


_ROWS = 1000


def _dis_of(degp_ref):
    deg = degp_ref[0, :, 0] + degp_ref[1, :, 0] + 1.0
    return lax.rsqrt(deg)[:, None]


def _prep_body(degp_ref, x_ref, w_ref, o1_ref, o2_ref, o1b_ref, o2b_ref):
    dis = _dis_of(degp_ref)
    h = jnp.dot(x_ref[...], w_ref[...],
                preferred_element_type=jnp.float32) * dis
    half = h.shape[1] // 2
    o1_ref[...] = h[:, :half]
    o2_ref[...] = h[:, half:]
    o1b_ref[...] = h[:, :half].astype(jnp.bfloat16)
    o2b_ref[...] = h[:, half:].astype(jnp.bfloat16)


def _tc_prep(degp, x, W):
    N, Din = x.shape
    Dh = W.shape[1]
    half = Dh // 2
    return pl.pallas_call(
        _prep_body,
        grid=(N // _ROWS,),
        in_specs=[
            pl.BlockSpec((NC, _ROWS, LANES), lambda i: (0, i, 0)),
            pl.BlockSpec((_ROWS, Din), lambda i: (i, 0)),
            pl.BlockSpec((Din, Dh), lambda i: (0, 0)),
        ],
        out_specs=[
            pl.BlockSpec((_ROWS, half), lambda i: (i, 0)),
            pl.BlockSpec((_ROWS, half), lambda i: (i, 0)),
            pl.BlockSpec((_ROWS, half), lambda i: (i, 0)),
            pl.BlockSpec((_ROWS, half), lambda i: (i, 0)),
        ],
        out_shape=[
            jax.ShapeDtypeStruct((N, half), jnp.float32),
            jax.ShapeDtypeStruct((N, half), jnp.float32),
            jax.ShapeDtypeStruct((N, half), jnp.bfloat16),
            jax.ShapeDtypeStruct((N, half), jnp.bfloat16),
        ],
    )(degp, x, W)


def _fp1_body(aggA_ref, aggB_ref, hpA_ref, hpB_ref, degp_ref, b_ref, w_ref,
              o_ref, ob_ref):
    dis = _dis_of(degp_ref)
    half = hpA_ref.shape[1]
    aA = (aggA_ref[0].astype(jnp.float32) + aggA_ref[1].astype(jnp.float32)
          + hpA_ref[...])
    aB = (aggB_ref[0].astype(jnp.float32) + aggB_ref[1].astype(jnp.float32)
          + hpB_ref[...])
    yA = jnp.maximum(dis * aA + b_ref[:, :half], 0.0)
    yB = jnp.maximum(dis * aB + b_ref[:, half:], 0.0)
    y = jnp.concatenate([yA, yB], axis=1)
    h = jnp.dot(y, w_ref[...], preferred_element_type=jnp.float32) * dis
    o_ref[...] = h
    ob_ref[...] = h.astype(jnp.bfloat16)


def _tc_finish_prep1(aggA, aggB, hpA, hpB, degp, b, W):
    N, half = hpA.shape
    D = 2 * half
    K = W.shape[1]
    return pl.pallas_call(
        _fp1_body,
        grid=(N // _ROWS,),
        in_specs=[
            pl.BlockSpec((NC, _ROWS, half), lambda i: (0, i, 0)),
            pl.BlockSpec((NC, _ROWS, half), lambda i: (0, i, 0)),
            pl.BlockSpec((_ROWS, half), lambda i: (i, 0)),
            pl.BlockSpec((_ROWS, half), lambda i: (i, 0)),
            pl.BlockSpec((NC, _ROWS, LANES), lambda i: (0, i, 0)),
            pl.BlockSpec((1, D), lambda i: (0, 0)),
            pl.BlockSpec((D, K), lambda i: (0, 0)),
        ],
        out_specs=[
            pl.BlockSpec((_ROWS, K), lambda i: (i, 0)),
            pl.BlockSpec((_ROWS, K), lambda i: (i, 0)),
        ],
        out_shape=[
            jax.ShapeDtypeStruct((N, K), jnp.float32),
            jax.ShapeDtypeStruct((N, K), jnp.bfloat16),
        ],
    )(aggA, aggB, hpA, hpB, degp, b.reshape(1, D), W)


def _fp_body(aggp_ref, hp_ref, degp_ref, b_ref, w_ref, o_ref, ob_ref):
    dis = _dis_of(degp_ref)
    a = (aggp_ref[0].astype(jnp.float32) + aggp_ref[1].astype(jnp.float32)
         + hp_ref[...])
    y = jnp.maximum(dis * a + b_ref[...], 0.0)
    h = jnp.dot(y, w_ref[...], preferred_element_type=jnp.float32) * dis
    o_ref[...] = h
    ob_ref[...] = h.astype(jnp.bfloat16)


def _tc_finish_prep(aggp, hp, degp, b, W, bf16_out=True):
    N, D = hp.shape
    K = W.shape[1]
    out_specs = [
        pl.BlockSpec((_ROWS, K), lambda i: (i, 0)),
        pl.BlockSpec((_ROWS, K), lambda i: (i, 0)),
    ]
    out_shape = [
        jax.ShapeDtypeStruct((N, K), jnp.float32),
        jax.ShapeDtypeStruct((N, K), jnp.bfloat16),
    ]
    body = _fp_body
    if not bf16_out:
        out_specs = out_specs[:1]
        out_shape = out_shape[:1]

        def body(aggp_ref, hp_ref, degp_ref, b_ref, w_ref, o_ref):
            _fp_body(aggp_ref, hp_ref, degp_ref, b_ref, w_ref, o_ref,
                     _NullRef())

    return pl.pallas_call(
        body,
        grid=(N // _ROWS,),
        in_specs=[
            pl.BlockSpec((NC, _ROWS, D), lambda i: (0, i, 0)),
            pl.BlockSpec((_ROWS, D), lambda i: (i, 0)),
            pl.BlockSpec((NC, _ROWS, LANES), lambda i: (0, i, 0)),
            pl.BlockSpec((1, D), lambda i: (0, 0)),
            pl.BlockSpec((D, K), lambda i: (0, 0)),
        ],
        out_specs=out_specs,
        out_shape=out_shape,
    )(aggp, hp, degp, b.reshape(1, D), W)


class _NullRef:
    def __setitem__(self, idx, val):
        pass


def _final_body(aggp_ref, hp_ref, degp_ref, b_ref, o_ref):
    dis = _dis_of(degp_ref)
    a = aggp_ref[0] + aggp_ref[1] + hp_ref[...]
    y = dis * a + b_ref[0, 0]
    o_ref[...] = y[:, :1]


def _tc_final(aggp, hp, degp, b):
    N = hp.shape[0]
    return pl.pallas_call(
        _final_body,
        grid=(N // _ROWS,),
        in_specs=[
            pl.BlockSpec((NC, _ROWS, LANES), lambda i: (0, i, 0)),
            pl.BlockSpec((_ROWS, LANES), lambda i: (i, 0)),
            pl.BlockSpec((NC, _ROWS, LANES), lambda i: (0, i, 0)),
            pl.BlockSpec((1, 1), lambda i: (0, 0)),
        ],
        out_specs=pl.BlockSpec((_ROWS, 1), lambda i: (i, 0)),
        out_shape=jax.ShapeDtypeStruct((N, 1), jnp.float32),
    )(aggp, hp, degp, b.reshape(1, 1))


def kernel(x, edge_list, W1, b1, W2, b2, W3, b3):
    N = x.shape[0]
    src = edge_list[0]
    dst = edge_list[1]
    W3p = jnp.pad(W3, ((0, 0), (0, LANES - W3.shape[1])))

    degp = _sc_agg(src, dst, hist_shape=(N, LANES))
    h1pA, h1pB, h1pAb, h1pBb = _tc_prep(degp, x, W1)
    agg1A = _sc_agg(src, dst, table=h1pAb)
    agg1B = _sc_agg(src, dst, table=h1pBb)
    h2p, h2pb = _tc_finish_prep1(agg1A, agg1B, h1pA, h1pB, degp, b1, W2)
    agg2 = _sc_agg(src, dst, table=h2pb)
    h3p = _tc_finish_prep(agg2, h2p, degp, b2, W3p,
                          bf16_out=False)[0]
    agg3 = _sc_agg(src, dst, table=h3p)
    return _tc_final(agg3, h3p, degp, b3)


# speedup vs baseline: 1.2679x; 1.0315x over previous
import functools

import jax
import jax.numpy as jnp
from jax import lax
from jax.experimental import pallas as pl
from jax.experimental.pallas import tpu as pltpu
from jax.experimental.pallas import tpu_sc as plsc

NC = 2
NS = 16
NW = NC * NS
LANES = 16


def _sc_agg(src, dst, table=None, hist_shape=None):
    gather = table is not None
    if gather:
        N, D = table.shape
        dtype = table.dtype
    else:
        N, D = hist_shape
        dtype = jnp.float32
    E = dst.shape[0]
    assert E % NW == 0
    EW = E // NW
    C = 80
    NB = 5
    assert EW % (C * NB) == 0
    nch = EW // C
    ng = nch // NB
    assert N % NS == 0
    TR = N // NS
    lanes = LANES * 4 // jnp.dtype(dtype).itemsize
    nv = D // lanes

    mesh = plsc.VectorSubcoreMesh(core_axis_name="c", subcore_axis_name="s")

    src3 = src.reshape(NW, nch, C) if gather else None
    dst3 = dst.reshape(NW, nch, C)

    scratch = [
        pltpu.VMEM((nch, C), jnp.int32),
        pltpu.VMEM((nch, C), jnp.int32),
        pltpu.VMEM((2, NB, C, D), dtype),
        pltpu.VMEM_SHARED((N, D), dtype),
        pltpu.SemaphoreType.DMA,
        pltpu.SemaphoreType.DMA,
    ]

    def body(table_h, src3_h, dst3_h, out_h, sidx, didx, bufs, acc, gsem, ssem):
        c = lax.axis_index("c")
        s = lax.axis_index("s")
        w = c * NS + s

        def fill_buf0(val):
            def fb(t, carry):
                r = t // nv
                col = (t % nv) * lanes
                bufs[0, 0, r, pl.ds(col, lanes)] = jnp.full(
                    (lanes,), val, dtype)
                return carry
            lax.fori_loop(0, C * nv, fb, 0)

        fill_buf0(0.0)
        r0 = s * TR
        off = 0
        while off < TR:
            m = min(C, TR - off)
            pltpu.sync_copy(bufs.at[0, 0, pl.ds(0, m)],
                            acc.at[pl.ds(r0 + off, m)])
            off += m

        pltpu.sync_copy(dst3_h.at[w], didx)
        if gather:
            pltpu.sync_copy(src3_h.at[w], sidx)
        else:
            fill_buf0(1.0)
        plsc.subcore_barrier()

        if gather:
            def start_gathers(g, p):
                for j in range(NB):
                    pltpu.async_copy(table_h.at[sidx.at[g * NB + j]],
                                     bufs.at[p, j], gsem)

            def drain(sem, p):
                for j in range(NB):
                    pltpu.make_async_copy(
                        table_h.at[pl.ds(0, C)], bufs.at[p, j], sem
                    ).wait()

            start_gathers(0, 0)

            def group(t, carry):
                p = lax.rem(t, 2)
                q = 1 - p
                drain(gsem, p)
                for j in range(NB):
                    pltpu.async_copy(bufs.at[p, j],
                                     acc.at[didx.at[t * NB + j]],
                                     ssem, add=True)

                @pl.when(t + 1 < ng)
                def _():
                    start_gathers(t + 1, q)

                drain(ssem, p)
                return carry

            lax.fori_loop(0, ng, group, 0)
        else:
            def group(t, carry):
                for j in range(NB):
                    pltpu.async_copy(bufs.at[0, 0],
                                     acc.at[didx.at[t * NB + j]],
                                     ssem, add=True)
                for j in range(NB):
                    pltpu.make_async_copy(
                        bufs.at[0, 0], acc.at[pl.ds(0, C)], ssem
                    ).wait()
                return carry

            lax.fori_loop(0, ng, group, 0)

        plsc.subcore_barrier()
        pltpu.sync_copy(acc.at[pl.ds(r0, TR)], out_h.at[c, s])

    if gather:
        args = (table, src3, dst3)

        def k_gather(table_h, src3_h, dst3_h, out_h,
                     sidx, didx, bufs, acc, gsem, ssem):
            body(table_h, src3_h, dst3_h, out_h,
                 sidx, didx, bufs, acc, gsem, ssem)

        fn = k_gather
    else:
        args = (dst3,)

        def k_hist(dst3_h, out_h, sidx, didx, bufs, acc, gsem, ssem):
            body(None, None, dst3_h, out_h,
                 sidx, didx, bufs, acc, gsem, ssem)

        fn = k_hist

    run = functools.partial(
        pl.kernel,
        mesh=mesh,
        out_type=jax.ShapeDtypeStruct((NC, NS, TR, D), dtype),
        scratch_types=scratch,
        compiler_params=pltpu.CompilerParams(use_tc_tiling_on_sc=False),
    )(fn)
    return run(*args).reshape(NC, N, D)


def _sc_agg_pair(src, dst, tableA, tableB):
    N, D = tableA.shape
    dtype = tableA.dtype
    assert tableB.shape == tableA.shape and tableB.dtype == dtype
    E = dst.shape[0]
    EW = E // NW
    C = 80
    NB = 5
    assert E % NW == 0 and EW % (C * NB) == 0 and N % NS == 0
    nch = EW // C
    ng = nch // NB
    TR = N // NS
    lanes = LANES * 4 // jnp.dtype(dtype).itemsize
    nv = D // lanes

    mesh = plsc.VectorSubcoreMesh(core_axis_name="c", subcore_axis_name="s")
    src3 = src.reshape(NW, nch, C)
    dst3 = dst.reshape(NW, nch, C)

    scratch = [
        pltpu.VMEM((nch, C), jnp.int32),
        pltpu.VMEM((nch, C), jnp.int32),
        pltpu.VMEM((2, NB, C, D), dtype),
        pltpu.VMEM((2, NB, C, D), dtype),
        pltpu.VMEM_SHARED((N, D), dtype),
        pltpu.VMEM_SHARED((N, D), dtype),
        pltpu.SemaphoreType.DMA,
        pltpu.SemaphoreType.DMA,
    ]

    @functools.partial(
        pl.kernel,
        mesh=mesh,
        out_type=[
            jax.ShapeDtypeStruct((NC, NS, TR, D), dtype),
            jax.ShapeDtypeStruct((NC, NS, TR, D), dtype),
        ],
        scratch_types=scratch,
        compiler_params=pltpu.CompilerParams(use_tc_tiling_on_sc=False),
    )
    def run(tabA_h, tabB_h, src3_h, dst3_h, outA_h, outB_h,
            sidx, didx, bufA, bufB, accA, accB, gsem, ssem):
        c = lax.axis_index("c")
        s = lax.axis_index("s")
        w = c * NS + s

        def zb(t, carry):
            r = t // nv
            col = (t % nv) * lanes
            bufA[0, 0, r, pl.ds(col, lanes)] = jnp.zeros((lanes,), dtype)
            return carry
        lax.fori_loop(0, C * nv, zb, 0)
        r0 = s * TR
        off = 0
        while off < TR:
            m = min(C, TR - off)
            pltpu.sync_copy(bufA.at[0, 0, pl.ds(0, m)],
                            accA.at[pl.ds(r0 + off, m)])
            pltpu.sync_copy(bufA.at[0, 0, pl.ds(0, m)],
                            accB.at[pl.ds(r0 + off, m)])
            off += m

        pltpu.sync_copy(dst3_h.at[w], didx)
        pltpu.sync_copy(src3_h.at[w], sidx)
        plsc.subcore_barrier()

        def start_gathers(g, p):
            for j in range(NB):
                pltpu.async_copy(tabA_h.at[sidx.at[g * NB + j]],
                                 bufA.at[p, j], gsem)
                pltpu.async_copy(tabB_h.at[sidx.at[g * NB + j]],
                                 bufB.at[p, j], gsem)

        def drain(sem, p):
            for j in range(NB):
                pltpu.make_async_copy(
                    tabA_h.at[pl.ds(0, C)], bufA.at[p, j], sem).wait()
                pltpu.make_async_copy(
                    tabB_h.at[pl.ds(0, C)], bufB.at[p, j], sem).wait()

        start_gathers(0, 0)

        def group(t, carry):
            p = lax.rem(t, 2)
            q = 1 - p
            drain(gsem, p)
            for j in range(NB):
                pltpu.async_copy(bufA.at[p, j], accA.at[didx.at[t * NB + j]],
                                 ssem, add=True)
                pltpu.async_copy(bufB.at[p, j], accB.at[didx.at[t * NB + j]],
                                 ssem, add=True)

            @pl.when(t + 1 < ng)
            def _():
                start_gathers(t + 1, q)

            drain(ssem, p)
            return carry

        lax.fori_loop(0, ng, group, 0)

        plsc.subcore_barrier()
        pltpu.sync_copy(accA.at[pl.ds(r0, TR)], outA_h.at[c, s])
        pltpu.sync_copy(accB.at[pl.ds(r0, TR)], outB_h.at[c, s])

    outA, outB = run(tableA, tableB, src3, dst3)
    return outA.reshape(NC, N, D), outB.reshape(NC, N, D)


_ROWS = 1000


def _dis_of(degp_ref):
    deg = degp_ref[0, :, 0] + degp_ref[1, :, 0] + 1.0
    return lax.rsqrt(deg)[:, None]


def _prep_body(degp_ref, x_ref, w_ref, o1_ref, o2_ref, o1b_ref, o2b_ref):
    dis = _dis_of(degp_ref)
    h = jnp.dot(x_ref[...], w_ref[...],
                preferred_element_type=jnp.float32) * dis
    half = h.shape[1] // 2
    o1_ref[...] = h[:, :half]
    o2_ref[...] = h[:, half:]
    o1b_ref[...] = h[:, :half].astype(jnp.bfloat16)
    o2b_ref[...] = h[:, half:].astype(jnp.bfloat16)


def _tc_prep(degp, x, W):
    N, Din = x.shape
    Dh = W.shape[1]
    half = Dh // 2
    return pl.pallas_call(
        _prep_body,
        grid=(N // _ROWS,),
        in_specs=[
            pl.BlockSpec((NC, _ROWS, LANES), lambda i: (0, i, 0)),
            pl.BlockSpec((_ROWS, Din), lambda i: (i, 0)),
            pl.BlockSpec((Din, Dh), lambda i: (0, 0)),
        ],
        out_specs=[
            pl.BlockSpec((_ROWS, half), lambda i: (i, 0)),
            pl.BlockSpec((_ROWS, half), lambda i: (i, 0)),
            pl.BlockSpec((_ROWS, half), lambda i: (i, 0)),
            pl.BlockSpec((_ROWS, half), lambda i: (i, 0)),
        ],
        out_shape=[
            jax.ShapeDtypeStruct((N, half), jnp.float32),
            jax.ShapeDtypeStruct((N, half), jnp.float32),
            jax.ShapeDtypeStruct((N, half), jnp.bfloat16),
            jax.ShapeDtypeStruct((N, half), jnp.bfloat16),
        ],
    )(degp, x, W)


def _fp1_body(aggA_ref, aggB_ref, hpA_ref, hpB_ref, degp_ref, b_ref, w_ref,
              o_ref, ob_ref):
    dis = _dis_of(degp_ref)
    half = hpA_ref.shape[1]
    aA = (aggA_ref[0].astype(jnp.float32) + aggA_ref[1].astype(jnp.float32)
          + hpA_ref[...])
    aB = (aggB_ref[0].astype(jnp.float32) + aggB_ref[1].astype(jnp.float32)
          + hpB_ref[...])
    yA = jnp.maximum(dis * aA + b_ref[:, :half], 0.0)
    yB = jnp.maximum(dis * aB + b_ref[:, half:], 0.0)
    y = jnp.concatenate([yA, yB], axis=1)
    h = jnp.dot(y, w_ref[...], preferred_element_type=jnp.float32) * dis
    o_ref[...] = h
    ob_ref[...] = h.astype(jnp.bfloat16)


def _tc_finish_prep1(aggA, aggB, hpA, hpB, degp, b, W):
    N, half = hpA.shape
    D = 2 * half
    K = W.shape[1]
    return pl.pallas_call(
        _fp1_body,
        grid=(N // _ROWS,),
        in_specs=[
            pl.BlockSpec((NC, _ROWS, half), lambda i: (0, i, 0)),
            pl.BlockSpec((NC, _ROWS, half), lambda i: (0, i, 0)),
            pl.BlockSpec((_ROWS, half), lambda i: (i, 0)),
            pl.BlockSpec((_ROWS, half), lambda i: (i, 0)),
            pl.BlockSpec((NC, _ROWS, LANES), lambda i: (0, i, 0)),
            pl.BlockSpec((1, D), lambda i: (0, 0)),
            pl.BlockSpec((D, K), lambda i: (0, 0)),
        ],
        out_specs=[
            pl.BlockSpec((_ROWS, K), lambda i: (i, 0)),
            pl.BlockSpec((_ROWS, K), lambda i: (i, 0)),
        ],
        out_shape=[
            jax.ShapeDtypeStruct((N, K), jnp.float32),
            jax.ShapeDtypeStruct((N, K), jnp.bfloat16),
        ],
    )(aggA, aggB, hpA, hpB, degp, b.reshape(1, D), W)


def _fp_body(aggp_ref, hp_ref, degp_ref, b_ref, w_ref, o_ref, ob_ref):
    dis = _dis_of(degp_ref)
    a = (aggp_ref[0].astype(jnp.float32) + aggp_ref[1].astype(jnp.float32)
         + hp_ref[...])
    y = jnp.maximum(dis * a + b_ref[...], 0.0)
    h = jnp.dot(y, w_ref[...], preferred_element_type=jnp.float32) * dis
    o_ref[...] = h
    ob_ref[...] = h.astype(jnp.bfloat16)


def _tc_finish_prep(aggp, hp, degp, b, W, bf16_out=True):
    N, D = hp.shape
    K = W.shape[1]
    out_specs = [
        pl.BlockSpec((_ROWS, K), lambda i: (i, 0)),
        pl.BlockSpec((_ROWS, K), lambda i: (i, 0)),
    ]
    out_shape = [
        jax.ShapeDtypeStruct((N, K), jnp.float32),
        jax.ShapeDtypeStruct((N, K), jnp.bfloat16),
    ]
    body = _fp_body
    if not bf16_out:
        out_specs = out_specs[:1]
        out_shape = out_shape[:1]

        def body(aggp_ref, hp_ref, degp_ref, b_ref, w_ref, o_ref):
            _fp_body(aggp_ref, hp_ref, degp_ref, b_ref, w_ref, o_ref,
                     _NullRef())

    return pl.pallas_call(
        body,
        grid=(N // _ROWS,),
        in_specs=[
            pl.BlockSpec((NC, _ROWS, D), lambda i: (0, i, 0)),
            pl.BlockSpec((_ROWS, D), lambda i: (i, 0)),
            pl.BlockSpec((NC, _ROWS, LANES), lambda i: (0, i, 0)),
            pl.BlockSpec((1, D), lambda i: (0, 0)),
            pl.BlockSpec((D, K), lambda i: (0, 0)),
        ],
        out_specs=out_specs,
        out_shape=out_shape,
    )(aggp, hp, degp, b.reshape(1, D), W)


class _NullRef:
    def __setitem__(self, idx, val):
        pass


def _final_body(aggp_ref, hp_ref, degp_ref, b_ref, o_ref):
    dis = _dis_of(degp_ref)
    a = aggp_ref[0] + aggp_ref[1] + hp_ref[...]
    y = dis * a + b_ref[0, 0]
    o_ref[...] = y[:, :1]


def _tc_final(aggp, hp, degp, b):
    N = hp.shape[0]
    return pl.pallas_call(
        _final_body,
        grid=(N // _ROWS,),
        in_specs=[
            pl.BlockSpec((NC, _ROWS, LANES), lambda i: (0, i, 0)),
            pl.BlockSpec((_ROWS, LANES), lambda i: (i, 0)),
            pl.BlockSpec((NC, _ROWS, LANES), lambda i: (0, i, 0)),
            pl.BlockSpec((1, 1), lambda i: (0, 0)),
        ],
        out_specs=pl.BlockSpec((_ROWS, 1), lambda i: (i, 0)),
        out_shape=jax.ShapeDtypeStruct((N, 1), jnp.float32),
    )(aggp, hp, degp, b.reshape(1, 1))


def kernel(x, edge_list, W1, b1, W2, b2, W3, b3):
    N = x.shape[0]
    src = edge_list[0]
    dst = edge_list[1]
    W3p = jnp.pad(W3, ((0, 0), (0, LANES - W3.shape[1])))

    degp = _sc_agg(src, dst, hist_shape=(N, LANES))
    h1pA, h1pB, h1pAb, h1pBb = _tc_prep(degp, x, W1)
    agg1A, agg1B = _sc_agg_pair(src, dst, h1pAb, h1pBb)
    h2p, h2pb = _tc_finish_prep1(agg1A, agg1B, h1pA, h1pB, degp, b1, W2)
    agg2 = _sc_agg(src, dst, table=h2pb)
    h3p = _tc_finish_prep(agg2, h2p, degp, b2, W3p,
                          bf16_out=False)[0]
    agg3 = _sc_agg(src, dst, table=h3p)
    return _tc_final(agg3, h3p, degp, b3)
